# baseline (speedup 1.0000x reference)
import functools

import jax
import jax.numpy as jnp
from jax import lax
from jax.experimental import pallas as pl
from jax.experimental.pallas import tpu as pltpu
from jax.experimental.pallas import tpu_sc as plsc

N = 10000
E = 160000
D_IN = 256
D_HID = 512
D_OUT = 8

NC = 2
NS = 16
CHUNK = 128
NBLK = 1280
E_PAD = NBLK * CHUNK
QW = D_IN // 4
P_W = 16
NROW = N + 8
ZROWS = 200
NZCHUNK = N // ZROWS
NBUF = 4
IDXH = 40


def _pipe(table, ei_v, agg_sh, rows_v, gsems, ssems, tpt):
    for b in range(NBUF):
        pltpu.async_copy(table.at[ei_v.at[0, b]], rows_v.at[b], gsems[b])

    @pl.loop(0, tpt // NBUF)
    def _(kk):
        for b in range(NBUF):
            t = NBUF * kk + b
            pltpu.make_async_copy(table.at[ei_v.at[0, t]],
                                  rows_v.at[b], gsems[b]).wait()
            pltpu.async_copy(rows_v.at[b], agg_sh.at[ei_v.at[1, t]],
                             ssems[b], add=True)
        for b in range(NBUF):
            t = NBUF * kk + b
            pltpu.make_async_copy(rows_v.at[b], agg_sh.at[ei_v.at[1, t]],
                                  ssems[b]).wait()

            @pl.when(kk < tpt // NBUF - 1)
            def _():
                pltpu.async_copy(table.at[ei_v.at[0, t + NBUF]],
                                 rows_v.at[b], gsems[b])


def _zero_init(sid, z_hbm, agg_sh):
    @pl.loop(sid, NZCHUNK, step=NS)
    def _(j):
        pltpu.sync_copy(z_hbm.at[pl.ds(0, ZROWS)],
                        agg_sh.at[pl.ds(j * ZROWS, ZROWS)])


def _writeback(sid, agg_sh, o_hbm):
    @pl.loop(sid, NZCHUNK, step=NS)
    def _(j):
        pltpu.sync_copy(agg_sh.at[pl.ds(j * ZROWS, ZROWS)],
                        o_hbm.at[pl.ds(j * ZROWS, ZROWS)])


def _seg_sum_l1(x, ei3, zrows):
    mesh = plsc.VectorSubcoreMesh(core_axis_name="c", subcore_axis_name="s")
    tpt = NBLK // NS

    @functools.partial(
        pl.kernel,
        mesh=mesh,
        compiler_params=pltpu.CompilerParams(use_tc_tiling_on_sc=False),
        out_type=[jax.ShapeDtypeStruct((N, QW), jnp.float32)
                  for _ in range(4)],
        scratch_types=[
            pltpu.VMEM((2, IDXH, CHUNK), jnp.int32),
            pltpu.VMEM((NBUF, CHUNK, QW), jnp.float32),
            pltpu.VMEM_SHARED((N, QW), jnp.float32),
            pltpu.VMEM_SHARED((NROW, QW), jnp.float32),
        ] + [pltpu.SemaphoreType.DMA for _ in range(2 * NBUF)],
    )
    def k(x_hbm, ei_hbm, z_hbm, o0_hbm, o1_hbm, o2_hbm, o3_hbm,
          ei_v, rows_v, x_sh, agg_sh, *sems):
        cid = lax.axis_index("c")
        sid = lax.axis_index("s")

        def one_pass(q, o_hbm):
            @pl.loop(sid, NZCHUNK, step=NS)
            def _(j):
                pltpu.sync_copy(
                    x_hbm.at[pl.ds(j * ZROWS, ZROWS), pl.ds(q * QW, QW)],
                    x_sh.at[pl.ds(j * ZROWS, ZROWS)])
                pltpu.sync_copy(z_hbm.at[pl.ds(0, ZROWS)],
                                agg_sh.at[pl.ds(j * ZROWS, ZROWS)])
            plsc.subcore_barrier()

            for h in range(tpt // IDXH):
                pltpu.sync_copy(
                    ei_hbm.at[:, pl.ds(sid * tpt + h * IDXH, IDXH), :], ei_v)
                _pipe(x_sh, ei_v, agg_sh, rows_v,
                      sems[:NBUF], sems[NBUF:], IDXH)
            plsc.subcore_barrier()
            _writeback(sid, agg_sh, o_hbm)
            plsc.subcore_barrier()

        @pl.when(cid == 0)
        def _():
            one_pass(0, o0_hbm)
            one_pass(1, o1_hbm)

        @pl.when(cid == 1)
        def _():
            one_pass(2, o2_hbm)
            one_pass(3, o3_hbm)

    return k(x, ei3, zrows)


def _seg_sum_l2(p, ei3, zrows):
    mesh = plsc.VectorSubcoreMesh(core_axis_name="c", subcore_axis_name="s")
    tpt = NBLK // (NC * NS)

    @functools.partial(
        pl.kernel,
        mesh=mesh,
        compiler_params=pltpu.CompilerParams(use_tc_tiling_on_sc=False),
        out_type=[jax.ShapeDtypeStruct((N, P_W), jnp.float32),
                  jax.ShapeDtypeStruct((N, P_W), jnp.float32)],
        scratch_types=[
            pltpu.VMEM((2, NBLK // (NC * NS), CHUNK), jnp.int32),
            pltpu.VMEM((NBUF, CHUNK, P_W), jnp.float32),
            pltpu.VMEM_SHARED((N, P_W), jnp.float32),
            pltpu.VMEM_SHARED((NROW, P_W), jnp.float32),
        ] + [pltpu.SemaphoreType.DMA for _ in range(2 * NBUF)],
    )
    def k(p_hbm, ei_hbm, z_hbm, oa_hbm, ob_hbm,
          ei_v, rows_v, p_sh, agg_sh, *sems):
        cid = lax.axis_index("c")
        sid = lax.axis_index("s")

        def run(lo_chunk, o_hbm):
            pltpu.sync_copy(
                ei_hbm.at[:, pl.ds(lo_chunk + sid * tpt, tpt), :], ei_v)

            @pl.loop(sid, NZCHUNK, step=NS)
            def _(j):
                pltpu.sync_copy(p_hbm.at[pl.ds(j * ZROWS, ZROWS)],
                                p_sh.at[pl.ds(j * ZROWS, ZROWS)])
                pltpu.sync_copy(z_hbm.at[pl.ds(0, ZROWS)],
                                agg_sh.at[pl.ds(j * ZROWS, ZROWS)])
            plsc.subcore_barrier()
            _pipe(p_sh, ei_v, agg_sh, rows_v, sems[:NBUF], sems[NBUF:], tpt)
            plsc.subcore_barrier()
            _writeback(sid, agg_sh, o_hbm)

        @pl.when(cid == 0)
        def _():
            run(0, oa_hbm)

        @pl.when(cid == 1)
        def _():
            run(NBLK // NC, ob_hbm)

    return k(p, ei3, zrows)


_R = 1000
ECHUNK = E // CHUNK
PADC = NBLK - ECHUNK


def _pad_edges(ei):
    def body(e_ref, o_ref):
        pad0 = jnp.zeros((1, PADC, CHUNK), jnp.int32)
        pad1 = jnp.full((1, PADC, CHUNK), N, jnp.int32)
        o_ref[...] = jnp.concatenate(
            [e_ref[...], jnp.concatenate([pad0, pad1], axis=0)], axis=1)

    return pl.pallas_call(
        body,
        grid=(1,),
        in_specs=[pl.BlockSpec((2, ECHUNK, CHUNK), lambda i: (0, 0, 0))],
        out_specs=pl.BlockSpec((2, NBLK, CHUNK), lambda i: (0, 0, 0)),
        out_shape=jax.ShapeDtypeStruct((2, NBLK, CHUNK), jnp.int32),
    )(ei)


def _tc_a(x, w1_top, b1):
    def body(x_ref, w_ref, b_ref, o_ref):
        o_ref[...] = jnp.dot(x_ref[...], w_ref[...],
                             preferred_element_type=jnp.float32) + b_ref[...]

    return pl.pallas_call(
        body,
        grid=(N // _R,),
        in_specs=[pl.BlockSpec((_R, D_IN), lambda i: (i, 0)),
                  pl.BlockSpec((D_IN, D_HID), lambda i: (0, 0)),
                  pl.BlockSpec((1, D_HID), lambda i: (0, 0))],
        out_specs=pl.BlockSpec((_R, D_HID), lambda i: (i, 0)),
        out_shape=jax.ShapeDtypeStruct((N, D_HID), jnp.float32),
    )(x, w1_top, b1.reshape(1, D_HID))


def _tc_b(u, aggs, w1_bot, w2_top, w2_bot_pad, b2):
    def body(u_ref, a0_ref, a1_ref, a2_ref, a3_ref, wb_ref, wt_ref, wp_ref,
             b2_ref, t_ref, p_ref):
        agg = jnp.concatenate(
            [a0_ref[...], a1_ref[...], a2_ref[...], a3_ref[...]],
            axis=1).astype(jnp.bfloat16)
        h = u_ref[...] + jnp.dot(agg, wb_ref[...].astype(jnp.bfloat16),
                                 preferred_element_type=jnp.float32)
        h = jnp.maximum(h, 0.0).astype(jnp.bfloat16)
        t_ref[...] = jnp.dot(h, wt_ref[...].astype(jnp.bfloat16),
                             preferred_element_type=jnp.float32) + b2_ref[...]
        p_ref[...] = jnp.dot(h, wp_ref[...].astype(jnp.bfloat16),
                             preferred_element_type=jnp.float32)

    return pl.pallas_call(
        body,
        grid=(N // _R,),
        in_specs=[pl.BlockSpec((_R, D_HID), lambda i: (i, 0))] +
                 [pl.BlockSpec((_R, QW), lambda i: (i, 0))
                  for _ in range(4)] +
                 [pl.BlockSpec((D_IN, D_HID), lambda i: (0, 0)),
                  pl.BlockSpec((D_HID, D_OUT), lambda i: (0, 0)),
                  pl.BlockSpec((D_HID, P_W), lambda i: (0, 0)),
                  pl.BlockSpec((1, D_OUT), lambda i: (0, 0))],
        out_specs=[pl.BlockSpec((_R, D_OUT), lambda i: (i, 0)),
                   pl.BlockSpec((_R, P_W), lambda i: (i, 0))],
        out_shape=[jax.ShapeDtypeStruct((N, D_OUT), jnp.float32),
                   jax.ShapeDtypeStruct((N, P_W), jnp.float32)],
    )(u, *aggs, w1_bot, w2_top, w2_bot_pad, b2.reshape(1, D_OUT))


def _tc_c(t, qa, qb):
    def body(t_ref, a_ref, b_ref, o_ref):
        o_ref[...] = t_ref[...] + a_ref[...] + b_ref[...]

    return pl.pallas_call(
        body,
        grid=(N // _R,),
        in_specs=[pl.BlockSpec((_R, D_OUT), lambda i: (i, 0)),
                  pl.BlockSpec((_R, D_OUT), lambda i: (i, 0)),
                  pl.BlockSpec((_R, D_OUT), lambda i: (i, 0))],
        out_specs=pl.BlockSpec((_R, D_OUT), lambda i: (i, 0)),
        out_shape=jax.ShapeDtypeStruct((N, D_OUT), jnp.float32),
    )(t, qa, qb)


def kernel(x, edge_index, W1, b1, W2, b2):
    ei3 = _pad_edges(edge_index.reshape(2, ECHUNK, CHUNK))
    z1 = jnp.zeros((ZROWS, QW), jnp.float32)
    z2 = jnp.zeros((ZROWS, P_W), jnp.float32)

    w1_top = W1[:D_IN]
    w1_bot = W1[D_IN:]
    w2_top = W2[:D_HID]
    w2_bot_pad = jnp.concatenate(
        [W2[D_HID:], jnp.zeros((D_HID, P_W - D_OUT), jnp.float32)], axis=1)

    aggs = _seg_sum_l1(x, ei3, z1)
    u = _tc_a(x, w1_top, b1)
    t, p = _tc_b(u, aggs, w1_bot, w2_top, w2_bot_pad, b2)
    pa, pb = _seg_sum_l2(p, ei3, z2)
    return _tc_c(t, pa[:, :D_OUT], pb[:, :D_OUT])


# speedup vs baseline: 1.0723x; 1.0723x over previous
import functools

import jax
import jax.numpy as jnp
from jax import lax
from jax.experimental import pallas as pl
from jax.experimental.pallas import tpu as pltpu
from jax.experimental.pallas import tpu_sc as plsc

N = 10000
E = 160000
D_IN = 256
D_HID = 512
D_OUT = 8

NC = 2
NS = 16
CHUNK = 128
NBLK = 1280
E_PAD = NBLK * CHUNK
QW = D_IN // 4
P_W = 16
NROW = N + 8
ZROWS = 200
NZCHUNK = N // ZROWS
NBUF = 4
IDXH = 40


def _pipe(table, ei_v, agg_sh, rows_v, gsems, ssems, tpt):
    del ssems
    for b in range(NBUF):
        pltpu.async_copy(table.at[ei_v.at[0, b]], rows_v.at[b], gsems[b])

    @pl.loop(0, tpt // NBUF)
    def _(kk):
        for b in range(NBUF):
            t = NBUF * kk + b
            pltpu.make_async_copy(table.at[ei_v.at[0, t]],
                                  rows_v.at[b], gsems[b]).wait()
            pltpu.sync_copy(rows_v.at[b], agg_sh.at[ei_v.at[1, t]], add=True)

            @pl.when(kk < tpt // NBUF - 1)
            def _():
                pltpu.async_copy(table.at[ei_v.at[0, t + NBUF]],
                                 rows_v.at[b], gsems[b])


def _zero_init(sid, z_hbm, agg_sh):
    @pl.loop(sid, NZCHUNK, step=NS)
    def _(j):
        pltpu.sync_copy(z_hbm.at[pl.ds(0, ZROWS)],
                        agg_sh.at[pl.ds(j * ZROWS, ZROWS)])


def _writeback(sid, agg_sh, o_hbm):
    @pl.loop(sid, NZCHUNK, step=NS)
    def _(j):
        pltpu.sync_copy(agg_sh.at[pl.ds(j * ZROWS, ZROWS)],
                        o_hbm.at[pl.ds(j * ZROWS, ZROWS)])


def _seg_sum_l1(x, ei3, zrows):
    mesh = plsc.VectorSubcoreMesh(core_axis_name="c", subcore_axis_name="s")
    tpt = NBLK // NS

    @functools.partial(
        pl.kernel,
        mesh=mesh,
        compiler_params=pltpu.CompilerParams(use_tc_tiling_on_sc=False),
        out_type=[jax.ShapeDtypeStruct((N, QW), jnp.float32)
                  for _ in range(4)],
        scratch_types=[
            pltpu.VMEM((2, IDXH, CHUNK), jnp.int32),
            pltpu.VMEM((NBUF, CHUNK, QW), jnp.float32),
            pltpu.VMEM_SHARED((N, QW), jnp.float32),
            pltpu.VMEM_SHARED((NROW, QW), jnp.float32),
        ] + [pltpu.SemaphoreType.DMA for _ in range(2 * NBUF)],
    )
    def k(x_hbm, ei_hbm, z_hbm, o0_hbm, o1_hbm, o2_hbm, o3_hbm,
          ei_v, rows_v, x_sh, agg_sh, *sems):
        cid = lax.axis_index("c")
        sid = lax.axis_index("s")

        def one_pass(q, o_hbm):
            @pl.loop(sid, NZCHUNK, step=NS)
            def _(j):
                pltpu.sync_copy(
                    x_hbm.at[pl.ds(j * ZROWS, ZROWS), pl.ds(q * QW, QW)],
                    x_sh.at[pl.ds(j * ZROWS, ZROWS)])
                pltpu.sync_copy(z_hbm.at[pl.ds(0, ZROWS)],
                                agg_sh.at[pl.ds(j * ZROWS, ZROWS)])
            plsc.subcore_barrier()

            for h in range(tpt // IDXH):
                pltpu.sync_copy(
                    ei_hbm.at[:, pl.ds(sid * tpt + h * IDXH, IDXH), :], ei_v)
                _pipe(x_sh, ei_v, agg_sh, rows_v,
                      sems[:NBUF], sems[NBUF:], IDXH)
            plsc.subcore_barrier()
            _writeback(sid, agg_sh, o_hbm)
            plsc.subcore_barrier()

        @pl.when(cid == 0)
        def _():
            one_pass(0, o0_hbm)
            one_pass(1, o1_hbm)

        @pl.when(cid == 1)
        def _():
            one_pass(2, o2_hbm)
            one_pass(3, o3_hbm)

    return k(x, ei3, zrows)


def _seg_sum_l2(p, ei3, zrows):
    mesh = plsc.VectorSubcoreMesh(core_axis_name="c", subcore_axis_name="s")
    tpt = NBLK // (NC * NS)

    @functools.partial(
        pl.kernel,
        mesh=mesh,
        compiler_params=pltpu.CompilerParams(use_tc_tiling_on_sc=False),
        out_type=[jax.ShapeDtypeStruct((N, P_W), jnp.float32),
                  jax.ShapeDtypeStruct((N, P_W), jnp.float32)],
        scratch_types=[
            pltpu.VMEM((2, NBLK // (NC * NS), CHUNK), jnp.int32),
            pltpu.VMEM((NBUF, CHUNK, P_W), jnp.float32),
            pltpu.VMEM_SHARED((N, P_W), jnp.float32),
            pltpu.VMEM_SHARED((NROW, P_W), jnp.float32),
        ] + [pltpu.SemaphoreType.DMA for _ in range(2 * NBUF)],
    )
    def k(p_hbm, ei_hbm, z_hbm, oa_hbm, ob_hbm,
          ei_v, rows_v, p_sh, agg_sh, *sems):
        cid = lax.axis_index("c")
        sid = lax.axis_index("s")

        def run(lo_chunk, o_hbm):
            pltpu.sync_copy(
                ei_hbm.at[:, pl.ds(lo_chunk + sid * tpt, tpt), :], ei_v)

            @pl.loop(sid, NZCHUNK, step=NS)
            def _(j):
                pltpu.sync_copy(p_hbm.at[pl.ds(j * ZROWS, ZROWS)],
                                p_sh.at[pl.ds(j * ZROWS, ZROWS)])
                pltpu.sync_copy(z_hbm.at[pl.ds(0, ZROWS)],
                                agg_sh.at[pl.ds(j * ZROWS, ZROWS)])
            plsc.subcore_barrier()
            _pipe(p_sh, ei_v, agg_sh, rows_v, sems[:NBUF], sems[NBUF:], tpt)
            plsc.subcore_barrier()
            _writeback(sid, agg_sh, o_hbm)

        @pl.when(cid == 0)
        def _():
            run(0, oa_hbm)

        @pl.when(cid == 1)
        def _():
            run(NBLK // NC, ob_hbm)

    return k(p, ei3, zrows)


_R = 1000
ECHUNK = E // CHUNK
PADC = NBLK - ECHUNK


def _pad_edges(ei):
    def body(e_ref, o_ref):
        pad0 = jnp.zeros((1, PADC, CHUNK), jnp.int32)
        pad1 = jnp.full((1, PADC, CHUNK), N, jnp.int32)
        o_ref[...] = jnp.concatenate(
            [e_ref[...], jnp.concatenate([pad0, pad1], axis=0)], axis=1)

    return pl.pallas_call(
        body,
        grid=(1,),
        in_specs=[pl.BlockSpec((2, ECHUNK, CHUNK), lambda i: (0, 0, 0))],
        out_specs=pl.BlockSpec((2, NBLK, CHUNK), lambda i: (0, 0, 0)),
        out_shape=jax.ShapeDtypeStruct((2, NBLK, CHUNK), jnp.int32),
    )(ei)


def _tc_a(x, w1_top, b1):
    def body(x_ref, w_ref, b_ref, o_ref):
        o_ref[...] = jnp.dot(x_ref[...], w_ref[...],
                             preferred_element_type=jnp.float32) + b_ref[...]

    return pl.pallas_call(
        body,
        grid=(N // _R,),
        in_specs=[pl.BlockSpec((_R, D_IN), lambda i: (i, 0)),
                  pl.BlockSpec((D_IN, D_HID), lambda i: (0, 0)),
                  pl.BlockSpec((1, D_HID), lambda i: (0, 0))],
        out_specs=pl.BlockSpec((_R, D_HID), lambda i: (i, 0)),
        out_shape=jax.ShapeDtypeStruct((N, D_HID), jnp.float32),
    )(x, w1_top, b1.reshape(1, D_HID))


def _tc_b(u, aggs, w1_bot, w2_top, w2_bot_pad, b2):
    def body(u_ref, a0_ref, a1_ref, a2_ref, a3_ref, wb_ref, wt_ref, wp_ref,
             b2_ref, t_ref, p_ref):
        agg = jnp.concatenate(
            [a0_ref[...], a1_ref[...], a2_ref[...], a3_ref[...]],
            axis=1).astype(jnp.bfloat16)
        h = u_ref[...] + jnp.dot(agg, wb_ref[...].astype(jnp.bfloat16),
                                 preferred_element_type=jnp.float32)
        h = jnp.maximum(h, 0.0).astype(jnp.bfloat16)
        t_ref[...] = jnp.dot(h, wt_ref[...].astype(jnp.bfloat16),
                             preferred_element_type=jnp.float32) + b2_ref[...]
        p_ref[...] = jnp.dot(h, wp_ref[...].astype(jnp.bfloat16),
                             preferred_element_type=jnp.float32)

    return pl.pallas_call(
        body,
        grid=(N // _R,),
        in_specs=[pl.BlockSpec((_R, D_HID), lambda i: (i, 0))] +
                 [pl.BlockSpec((_R, QW), lambda i: (i, 0))
                  for _ in range(4)] +
                 [pl.BlockSpec((D_IN, D_HID), lambda i: (0, 0)),
                  pl.BlockSpec((D_HID, D_OUT), lambda i: (0, 0)),
                  pl.BlockSpec((D_HID, P_W), lambda i: (0, 0)),
                  pl.BlockSpec((1, D_OUT), lambda i: (0, 0))],
        out_specs=[pl.BlockSpec((_R, D_OUT), lambda i: (i, 0)),
                   pl.BlockSpec((_R, P_W), lambda i: (i, 0))],
        out_shape=[jax.ShapeDtypeStruct((N, D_OUT), jnp.float32),
                   jax.ShapeDtypeStruct((N, P_W), jnp.float32)],
    )(u, *aggs, w1_bot, w2_top, w2_bot_pad, b2.reshape(1, D_OUT))


def _tc_c(t, qa, qb):
    def body(t_ref, a_ref, b_ref, o_ref):
        o_ref[...] = t_ref[...] + a_ref[...] + b_ref[...]

    return pl.pallas_call(
        body,
        grid=(N // _R,),
        in_specs=[pl.BlockSpec((_R, D_OUT), lambda i: (i, 0)),
                  pl.BlockSpec((_R, D_OUT), lambda i: (i, 0)),
                  pl.BlockSpec((_R, D_OUT), lambda i: (i, 0))],
        out_specs=pl.BlockSpec((_R, D_OUT), lambda i: (i, 0)),
        out_shape=jax.ShapeDtypeStruct((N, D_OUT), jnp.float32),
    )(t, qa, qb)


def kernel(x, edge_index, W1, b1, W2, b2):
    ei3 = _pad_edges(edge_index.reshape(2, ECHUNK, CHUNK))
    z1 = jnp.zeros((ZROWS, QW), jnp.float32)
    z2 = jnp.zeros((ZROWS, P_W), jnp.float32)

    w1_top = W1[:D_IN]
    w1_bot = W1[D_IN:]
    w2_top = W2[:D_HID]
    w2_bot_pad = jnp.concatenate(
        [W2[D_HID:], jnp.zeros((D_HID, P_W - D_OUT), jnp.float32)], axis=1)

    aggs = _seg_sum_l1(x, ei3, z1)
    u = _tc_a(x, w1_top, b1)
    t, p = _tc_b(u, aggs, w1_bot, w2_top, w2_bot_pad, b2)
    pa, pb = _seg_sum_l2(p, ei3, z2)
    return _tc_c(t, pa[:, :D_OUT], pb[:, :D_OUT])


# speedup vs baseline: 1.1923x; 1.1119x over previous
import functools

import jax
import jax.numpy as jnp
from jax import lax
from jax.experimental import pallas as pl
from jax.experimental.pallas import tpu as pltpu
from jax.experimental.pallas import tpu_sc as plsc

N = 10000
E = 160000
D_IN = 256
D_HID = 512
D_OUT = 8

NC = 2
NS = 16
CHUNK = 128
NBLK = 1280
E_PAD = NBLK * CHUNK
QW = D_IN // 4
P_W = 16
NROW = N + 8
ZROWS = 200
NZCHUNK = N // ZROWS
NBUF = 4
IDXH = 40


def _pipe(table, ei_v, agg_sh, rows_v, gsems, ssems, tpt):
    del ssems
    for b in range(NBUF):
        pltpu.async_copy(table.at[ei_v.at[0, b]], rows_v.at[b], gsems[b])

    @pl.loop(0, tpt // NBUF)
    def _(kk):
        for b in range(NBUF):
            t = NBUF * kk + b
            pltpu.make_async_copy(table.at[ei_v.at[0, t]],
                                  rows_v.at[b], gsems[b]).wait()
            pltpu.sync_copy(rows_v.at[b], agg_sh.at[ei_v.at[1, t]], add=True)

            @pl.when(kk < tpt // NBUF - 1)
            def _():
                pltpu.async_copy(table.at[ei_v.at[0, t + NBUF]],
                                 rows_v.at[b], gsems[b])


def _zero_init(sid, z_hbm, agg_sh):
    @pl.loop(sid, NZCHUNK, step=NS)
    def _(j):
        pltpu.sync_copy(z_hbm.at[pl.ds(0, ZROWS)],
                        agg_sh.at[pl.ds(j * ZROWS, ZROWS)])


def _writeback(sid, agg_sh, o_hbm):
    @pl.loop(sid, NZCHUNK, step=NS)
    def _(j):
        pltpu.sync_copy(agg_sh.at[pl.ds(j * ZROWS, ZROWS)],
                        o_hbm.at[pl.ds(j * ZROWS, ZROWS)])


def _seg_sum_l1(x, ei3, zrows):
    mesh = plsc.VectorSubcoreMesh(core_axis_name="c", subcore_axis_name="s")
    tpt = NBLK // NS

    @functools.partial(
        pl.kernel,
        mesh=mesh,
        compiler_params=pltpu.CompilerParams(use_tc_tiling_on_sc=False),
        out_type=[jax.ShapeDtypeStruct((N, 2 * QW), jnp.float32)
                  for _ in range(2)],
        scratch_types=[
            pltpu.VMEM((2, IDXH, CHUNK), jnp.int32),
            pltpu.VMEM((NBUF, CHUNK, QW), jnp.float32),
            pltpu.VMEM_SHARED((N, QW), jnp.float32),
            pltpu.VMEM_SHARED((NROW, QW), jnp.float32),
        ] + [pltpu.SemaphoreType.DMA for _ in range(2 * NBUF)],
    )
    def k(x_hbm, ei_hbm, z_hbm, o0_hbm, o1_hbm,
          ei_v, rows_v, x_sh, agg_sh, *sems):
        cid = lax.axis_index("c")
        sid = lax.axis_index("s")

        def one_pass(q, o_hbm, col):
            @pl.loop(sid, NZCHUNK, step=NS)
            def _(j):
                pltpu.sync_copy(
                    x_hbm.at[pl.ds(j * ZROWS, ZROWS), pl.ds(q * QW, QW)],
                    x_sh.at[pl.ds(j * ZROWS, ZROWS)])
                pltpu.sync_copy(z_hbm.at[pl.ds(0, ZROWS)],
                                agg_sh.at[pl.ds(j * ZROWS, ZROWS)])
            plsc.subcore_barrier()

            for h in range(tpt // IDXH):
                pltpu.sync_copy(
                    ei_hbm.at[:, pl.ds(sid * tpt + h * IDXH, IDXH), :], ei_v)
                _pipe(x_sh, ei_v, agg_sh, rows_v,
                      sems[:NBUF], sems[NBUF:], IDXH)
            plsc.subcore_barrier()

            @pl.loop(sid, NZCHUNK, step=NS)
            def _(j):
                pltpu.sync_copy(
                    agg_sh.at[pl.ds(j * ZROWS, ZROWS)],
                    o_hbm.at[pl.ds(j * ZROWS, ZROWS), pl.ds(col, QW)])
            plsc.subcore_barrier()

        @pl.when(cid == 0)
        def _():
            one_pass(0, o0_hbm, 0)
            one_pass(1, o0_hbm, QW)

        @pl.when(cid == 1)
        def _():
            one_pass(2, o1_hbm, 0)
            one_pass(3, o1_hbm, QW)

    return k(x, ei3, zrows)


def _seg_sum_l2(p, ei3, zrows):
    mesh = plsc.VectorSubcoreMesh(core_axis_name="c", subcore_axis_name="s")
    tpt = NBLK // (NC * NS)

    @functools.partial(
        pl.kernel,
        mesh=mesh,
        compiler_params=pltpu.CompilerParams(use_tc_tiling_on_sc=False),
        out_type=jax.ShapeDtypeStruct((N, 128), jnp.float32),
        scratch_types=[
            pltpu.VMEM((2, NBLK // (NC * NS), CHUNK), jnp.int32),
            pltpu.VMEM((NBUF, CHUNK, P_W), jnp.float32),
            pltpu.VMEM_SHARED((N, P_W), jnp.float32),
            pltpu.VMEM_SHARED((NROW, P_W), jnp.float32),
        ] + [pltpu.SemaphoreType.DMA for _ in range(2 * NBUF)],
    )
    def k(p_hbm, ei_hbm, z_hbm, o_hbm,
          ei_v, rows_v, p_sh, agg_sh, *sems):
        cid = lax.axis_index("c")
        sid = lax.axis_index("s")

        def run(lo_chunk, col):
            pltpu.sync_copy(
                ei_hbm.at[:, pl.ds(lo_chunk + sid * tpt, tpt), :], ei_v)

            @pl.loop(sid, NZCHUNK, step=NS)
            def _(j):
                pltpu.sync_copy(
                    p_hbm.at[pl.ds(j * ZROWS, ZROWS), pl.ds(0, P_W)],
                    p_sh.at[pl.ds(j * ZROWS, ZROWS)])
                pltpu.sync_copy(z_hbm.at[pl.ds(0, ZROWS)],
                                agg_sh.at[pl.ds(j * ZROWS, ZROWS)])
            plsc.subcore_barrier()
            _pipe(p_sh, ei_v, agg_sh, rows_v, sems[:NBUF], sems[NBUF:], tpt)
            plsc.subcore_barrier()

            @pl.loop(sid, NZCHUNK, step=NS)
            def _(j):
                pltpu.sync_copy(
                    agg_sh.at[pl.ds(j * ZROWS, ZROWS)],
                    o_hbm.at[pl.ds(j * ZROWS, ZROWS), pl.ds(col, P_W)])

        @pl.when(cid == 0)
        def _():
            run(0, 0)

        @pl.when(cid == 1)
        def _():
            run(NBLK // NC, P_W)

    return k(p, ei3, zrows)


_R = 1000
ECHUNK = E // CHUNK
PADC = NBLK - ECHUNK


def _pad_edges(ei):
    def body(e_ref, o_ref):
        pad0 = jnp.zeros((1, PADC, CHUNK), jnp.int32)
        pad1 = jnp.full((1, PADC, CHUNK), N, jnp.int32)
        o_ref[...] = jnp.concatenate(
            [e_ref[...], jnp.concatenate([pad0, pad1], axis=0)], axis=1)

    return pl.pallas_call(
        body,
        grid=(1,),
        in_specs=[pl.BlockSpec((2, ECHUNK, CHUNK), lambda i: (0, 0, 0))],
        out_specs=pl.BlockSpec((2, NBLK, CHUNK), lambda i: (0, 0, 0)),
        out_shape=jax.ShapeDtypeStruct((2, NBLK, CHUNK), jnp.int32),
    )(ei)


def _tc_a(x, w1_top, b1):
    def body(x_ref, w_ref, b_ref, o_ref):
        o_ref[...] = (jnp.dot(x_ref[...], w_ref[...],
                              preferred_element_type=jnp.float32)
                      + b_ref[...]).astype(jnp.bfloat16)

    return pl.pallas_call(
        body,
        grid=(N // _R,),
        in_specs=[pl.BlockSpec((_R, D_IN), lambda i: (i, 0)),
                  pl.BlockSpec((D_IN, D_HID), lambda i: (0, 0)),
                  pl.BlockSpec((1, D_HID), lambda i: (0, 0))],
        out_specs=pl.BlockSpec((_R, D_HID), lambda i: (i, 0)),
        out_shape=jax.ShapeDtypeStruct((N, D_HID), jnp.bfloat16),
    )(x, w1_top, b1.reshape(1, D_HID))


def _tc_b(u, a0, a1, w1_bot, w2_top, w2_bot_pad, b2):
    def body(u_ref, a0_ref, a1_ref, wb_ref, wt_ref, wp_ref,
             b2_ref, t_ref, p_ref):
        agg = jnp.concatenate(
            [a0_ref[...], a1_ref[...]], axis=1).astype(jnp.bfloat16)
        h = (u_ref[...].astype(jnp.float32)
             + jnp.dot(agg, wb_ref[...].astype(jnp.bfloat16),
                       preferred_element_type=jnp.float32))
        h = jnp.maximum(h, 0.0).astype(jnp.bfloat16)
        t_ref[...] = jnp.dot(h, wt_ref[...].astype(jnp.bfloat16),
                             preferred_element_type=jnp.float32) + b2_ref[...]
        p_ref[...] = jnp.dot(h, wp_ref[...].astype(jnp.bfloat16),
                             preferred_element_type=jnp.float32)

    return pl.pallas_call(
        body,
        grid=(N // _R,),
        in_specs=[pl.BlockSpec((_R, D_HID), lambda i: (i, 0)),
                  pl.BlockSpec((_R, 2 * QW), lambda i: (i, 0)),
                  pl.BlockSpec((_R, 2 * QW), lambda i: (i, 0)),
                  pl.BlockSpec((D_IN, D_HID), lambda i: (0, 0)),
                  pl.BlockSpec((D_HID, D_OUT), lambda i: (0, 0)),
                  pl.BlockSpec((D_HID, 128), lambda i: (0, 0)),
                  pl.BlockSpec((1, D_OUT), lambda i: (0, 0))],
        out_specs=[pl.BlockSpec((_R, D_OUT), lambda i: (i, 0)),
                   pl.BlockSpec((_R, 128), lambda i: (i, 0))],
        out_shape=[jax.ShapeDtypeStruct((N, D_OUT), jnp.float32),
                   jax.ShapeDtypeStruct((N, 128), jnp.float32)],
    )(u, a0, a1, w1_bot, w2_top, w2_bot_pad, b2.reshape(1, D_OUT))


def _tc_c(t, q):
    def body(t_ref, q_ref, o_ref):
        q = q_ref[...]
        o_ref[...] = t_ref[...] + q[:, :D_OUT] + q[:, P_W:P_W + D_OUT]

    return pl.pallas_call(
        body,
        grid=(N // _R,),
        in_specs=[pl.BlockSpec((_R, D_OUT), lambda i: (i, 0)),
                  pl.BlockSpec((_R, 128), lambda i: (i, 0))],
        out_specs=pl.BlockSpec((_R, D_OUT), lambda i: (i, 0)),
        out_shape=jax.ShapeDtypeStruct((N, D_OUT), jnp.float32),
    )(t, q)


def kernel(x, edge_index, W1, b1, W2, b2):
    ei3 = _pad_edges(edge_index.reshape(2, ECHUNK, CHUNK))
    z1 = jnp.zeros((ZROWS, QW), jnp.float32)
    z2 = jnp.zeros((ZROWS, P_W), jnp.float32)

    w1_top = W1[:D_IN]
    w1_bot = W1[D_IN:]
    w2_top = W2[:D_HID]
    w2_bot_pad = jnp.concatenate(
        [W2[D_HID:], jnp.zeros((D_HID, 128 - D_OUT), jnp.float32)], axis=1)

    a0, a1 = _seg_sum_l1(x, ei3, z1)
    u = _tc_a(x, w1_top, b1)
    t, p = _tc_b(u, a0, a1, w1_bot, w2_top, w2_bot_pad, b2)
    q = _seg_sum_l2(p, ei3, z2)
    return _tc_c(t, q)


# speedup vs baseline: 1.1953x; 1.0025x over previous
import functools

import jax
import jax.numpy as jnp
from jax import lax
from jax.experimental import pallas as pl
from jax.experimental.pallas import tpu as pltpu
from jax.experimental.pallas import tpu_sc as plsc

N = 10000
E = 160000
D_IN = 256
D_HID = 512
D_OUT = 8

NC = 2
NS = 16
CHUNK = 128
NBLK = 1280
E_PAD = NBLK * CHUNK
QW = D_IN // 4
P_W = 16
NROW = N + 8
ZROWS = 200
NZCHUNK = N // ZROWS
NBUF = 4
NBUF2 = 8
IDXH = 40


def _pipe(table, ei_v, agg_sh, rows_v, gsems, ssems, tpt, nbuf=NBUF):
    del ssems
    for b in range(nbuf):
        pltpu.async_copy(table.at[ei_v.at[0, b]], rows_v.at[b], gsems[b])

    @pl.loop(0, tpt // nbuf)
    def _(kk):
        for b in range(nbuf):
            t = nbuf * kk + b
            pltpu.make_async_copy(table.at[ei_v.at[0, t]],
                                  rows_v.at[b], gsems[b]).wait()
            pltpu.sync_copy(rows_v.at[b], agg_sh.at[ei_v.at[1, t]], add=True)

            @pl.when(kk < tpt // nbuf - 1)
            def _():
                pltpu.async_copy(table.at[ei_v.at[0, t + nbuf]],
                                 rows_v.at[b], gsems[b])


def _zero_init(sid, z_hbm, agg_sh):
    @pl.loop(sid, NZCHUNK, step=NS)
    def _(j):
        pltpu.sync_copy(z_hbm.at[pl.ds(0, ZROWS)],
                        agg_sh.at[pl.ds(j * ZROWS, ZROWS)])


def _writeback(sid, agg_sh, o_hbm):
    @pl.loop(sid, NZCHUNK, step=NS)
    def _(j):
        pltpu.sync_copy(agg_sh.at[pl.ds(j * ZROWS, ZROWS)],
                        o_hbm.at[pl.ds(j * ZROWS, ZROWS)])


def _seg_sum_l1(x, ei3, zrows):
    mesh = plsc.VectorSubcoreMesh(core_axis_name="c", subcore_axis_name="s")
    tpt = NBLK // NS

    @functools.partial(
        pl.kernel,
        mesh=mesh,
        compiler_params=pltpu.CompilerParams(use_tc_tiling_on_sc=False),
        out_type=[jax.ShapeDtypeStruct((N, 2 * QW), jnp.float32)
                  for _ in range(2)],
        scratch_types=[
            pltpu.VMEM((2, IDXH, CHUNK), jnp.int32),
            pltpu.VMEM((NBUF, CHUNK, QW), jnp.float32),
            pltpu.VMEM_SHARED((N, QW), jnp.float32),
            pltpu.VMEM_SHARED((NROW, QW), jnp.float32),
        ] + [pltpu.SemaphoreType.DMA for _ in range(2 * NBUF)],
    )
    def k(x_hbm, ei_hbm, z_hbm, o0_hbm, o1_hbm,
          ei_v, rows_v, x_sh, agg_sh, *sems):
        cid = lax.axis_index("c")
        sid = lax.axis_index("s")

        def one_pass(q, o_hbm, col):
            @pl.loop(sid, NZCHUNK, step=NS)
            def _(j):
                pltpu.sync_copy(
                    x_hbm.at[pl.ds(j * ZROWS, ZROWS), pl.ds(q * QW, QW)],
                    x_sh.at[pl.ds(j * ZROWS, ZROWS)])
                pltpu.sync_copy(z_hbm.at[pl.ds(0, ZROWS)],
                                agg_sh.at[pl.ds(j * ZROWS, ZROWS)])
            plsc.subcore_barrier()

            for h in range(tpt // IDXH):
                pltpu.sync_copy(
                    ei_hbm.at[:, pl.ds(sid * tpt + h * IDXH, IDXH), :], ei_v)
                _pipe(x_sh, ei_v, agg_sh, rows_v,
                      sems[:NBUF], sems[NBUF:], IDXH)
            plsc.subcore_barrier()

            @pl.loop(sid, NZCHUNK, step=NS)
            def _(j):
                pltpu.sync_copy(
                    agg_sh.at[pl.ds(j * ZROWS, ZROWS)],
                    o_hbm.at[pl.ds(j * ZROWS, ZROWS), pl.ds(col, QW)])
            plsc.subcore_barrier()

        @pl.when(cid == 0)
        def _():
            one_pass(0, o0_hbm, 0)
            one_pass(1, o0_hbm, QW)

        @pl.when(cid == 1)
        def _():
            one_pass(2, o1_hbm, 0)
            one_pass(3, o1_hbm, QW)

    return k(x, ei3, zrows)


def _seg_sum_l2(p, ei3, zrows):
    mesh = plsc.VectorSubcoreMesh(core_axis_name="c", subcore_axis_name="s")
    tpt = NBLK // (NC * NS)

    @functools.partial(
        pl.kernel,
        mesh=mesh,
        compiler_params=pltpu.CompilerParams(use_tc_tiling_on_sc=False),
        out_type=jax.ShapeDtypeStruct((N, 128), jnp.float32),
        scratch_types=[
            pltpu.VMEM((2, NBLK // (NC * NS), CHUNK), jnp.int32),
            pltpu.VMEM((NBUF2, CHUNK, P_W), jnp.float32),
            pltpu.VMEM_SHARED((N, P_W), jnp.float32),
            pltpu.VMEM_SHARED((NROW, P_W), jnp.float32),
        ] + [pltpu.SemaphoreType.DMA for _ in range(2 * NBUF2)],
    )
    def k(p_hbm, ei_hbm, z_hbm, o_hbm,
          ei_v, rows_v, p_sh, agg_sh, *sems):
        cid = lax.axis_index("c")
        sid = lax.axis_index("s")

        def run(lo_chunk, col):
            pltpu.sync_copy(
                ei_hbm.at[:, pl.ds(lo_chunk + sid * tpt, tpt), :], ei_v)

            @pl.loop(sid, NZCHUNK, step=NS)
            def _(j):
                pltpu.sync_copy(
                    p_hbm.at[pl.ds(j * ZROWS, ZROWS), pl.ds(0, P_W)],
                    p_sh.at[pl.ds(j * ZROWS, ZROWS)])
                pltpu.sync_copy(z_hbm.at[pl.ds(0, ZROWS)],
                                agg_sh.at[pl.ds(j * ZROWS, ZROWS)])
            plsc.subcore_barrier()
            _pipe(p_sh, ei_v, agg_sh, rows_v, sems[:NBUF2], sems[NBUF2:],
                  tpt, NBUF2)
            plsc.subcore_barrier()

            @pl.loop(sid, NZCHUNK, step=NS)
            def _(j):
                pltpu.sync_copy(
                    agg_sh.at[pl.ds(j * ZROWS, ZROWS)],
                    o_hbm.at[pl.ds(j * ZROWS, ZROWS), pl.ds(col, P_W)])

        @pl.when(cid == 0)
        def _():
            run(0, 0)

        @pl.when(cid == 1)
        def _():
            run(NBLK // NC, P_W)

    return k(p, ei3, zrows)


_R = 1000
ECHUNK = E // CHUNK
PADC = NBLK - ECHUNK


def _pad_edges(ei):
    def body(e_ref, o_ref):
        pad0 = jnp.zeros((1, PADC, CHUNK), jnp.int32)
        pad1 = jnp.full((1, PADC, CHUNK), N, jnp.int32)
        o_ref[...] = jnp.concatenate(
            [e_ref[...], jnp.concatenate([pad0, pad1], axis=0)], axis=1)

    return pl.pallas_call(
        body,
        grid=(1,),
        in_specs=[pl.BlockSpec((2, ECHUNK, CHUNK), lambda i: (0, 0, 0))],
        out_specs=pl.BlockSpec((2, NBLK, CHUNK), lambda i: (0, 0, 0)),
        out_shape=jax.ShapeDtypeStruct((2, NBLK, CHUNK), jnp.int32),
    )(ei)


def _tc_a(x, w1_top, b1):
    def body(x_ref, w_ref, b_ref, o_ref):
        o_ref[...] = (jnp.dot(x_ref[...], w_ref[...],
                              preferred_element_type=jnp.float32)
                      + b_ref[...]).astype(jnp.bfloat16)

    return pl.pallas_call(
        body,
        grid=(N // _R,),
        in_specs=[pl.BlockSpec((_R, D_IN), lambda i: (i, 0)),
                  pl.BlockSpec((D_IN, D_HID), lambda i: (0, 0)),
                  pl.BlockSpec((1, D_HID), lambda i: (0, 0))],
        out_specs=pl.BlockSpec((_R, D_HID), lambda i: (i, 0)),
        out_shape=jax.ShapeDtypeStruct((N, D_HID), jnp.bfloat16),
    )(x, w1_top, b1.reshape(1, D_HID))


def _tc_b(u, a0, a1, w1_bot, w2_top, w2_bot_pad, b2):
    def body(u_ref, a0_ref, a1_ref, wb_ref, wt_ref, wp_ref,
             b2_ref, t_ref, p_ref):
        agg = jnp.concatenate(
            [a0_ref[...], a1_ref[...]], axis=1).astype(jnp.bfloat16)
        h = (u_ref[...].astype(jnp.float32)
             + jnp.dot(agg, wb_ref[...].astype(jnp.bfloat16),
                       preferred_element_type=jnp.float32))
        h = jnp.maximum(h, 0.0).astype(jnp.bfloat16)
        t_ref[...] = jnp.dot(h, wt_ref[...].astype(jnp.bfloat16),
                             preferred_element_type=jnp.float32) + b2_ref[...]
        p_ref[...] = jnp.dot(h, wp_ref[...].astype(jnp.bfloat16),
                             preferred_element_type=jnp.float32)

    return pl.pallas_call(
        body,
        grid=(N // _R,),
        in_specs=[pl.BlockSpec((_R, D_HID), lambda i: (i, 0)),
                  pl.BlockSpec((_R, 2 * QW), lambda i: (i, 0)),
                  pl.BlockSpec((_R, 2 * QW), lambda i: (i, 0)),
                  pl.BlockSpec((D_IN, D_HID), lambda i: (0, 0)),
                  pl.BlockSpec((D_HID, D_OUT), lambda i: (0, 0)),
                  pl.BlockSpec((D_HID, 128), lambda i: (0, 0)),
                  pl.BlockSpec((1, D_OUT), lambda i: (0, 0))],
        out_specs=[pl.BlockSpec((_R, D_OUT), lambda i: (i, 0)),
                   pl.BlockSpec((_R, 128), lambda i: (i, 0))],
        out_shape=[jax.ShapeDtypeStruct((N, D_OUT), jnp.float32),
                   jax.ShapeDtypeStruct((N, 128), jnp.float32)],
    )(u, a0, a1, w1_bot, w2_top, w2_bot_pad, b2.reshape(1, D_OUT))


def _tc_c(t, q):
    def body(t_ref, q_ref, o_ref):
        q = q_ref[...]
        o_ref[...] = t_ref[...] + q[:, :D_OUT] + q[:, P_W:P_W + D_OUT]

    return pl.pallas_call(
        body,
        grid=(N // _R,),
        in_specs=[pl.BlockSpec((_R, D_OUT), lambda i: (i, 0)),
                  pl.BlockSpec((_R, 128), lambda i: (i, 0))],
        out_specs=pl.BlockSpec((_R, D_OUT), lambda i: (i, 0)),
        out_shape=jax.ShapeDtypeStruct((N, D_OUT), jnp.float32),
    )(t, q)


def kernel(x, edge_index, W1, b1, W2, b2):
    ei3 = _pad_edges(edge_index.reshape(2, ECHUNK, CHUNK))
    z1 = jnp.zeros((ZROWS, QW), jnp.float32)
    z2 = jnp.zeros((ZROWS, P_W), jnp.float32)

    w1_top = W1[:D_IN]
    w1_bot = W1[D_IN:]
    w2_top = W2[:D_HID]
    w2_bot_pad = jnp.concatenate(
        [W2[D_HID:], jnp.zeros((D_HID, 128 - D_OUT), jnp.float32)], axis=1)

    a0, a1 = _seg_sum_l1(x, ei3, z1)
    u = _tc_a(x, w1_top, b1)
    t, p = _tc_b(u, a0, a1, w1_bot, w2_top, w2_bot_pad, b2)
    q = _seg_sum_l2(p, ei3, z2)
    return _tc_c(t, q)
